# baseline (speedup 1.0000x reference)
import jax
import jax.numpy as jnp
from jax.experimental import pallas as pl
from jax.experimental.pallas import tpu as pltpu
from jax.experimental.pallas import tpu_sc as plsc

HIDDEN = 128
EMBED = 1024
GATHER_WINDOW = 128
MM_BLK = 2048
CHUNKS = 4


def _sc_gather(table, ids_flat):
    n = ids_flat.shape[0]
    ids2 = ids_flat.reshape(1, n)
    mesh = plsc.VectorSubcoreMesh(core_axis_name="core", subcore_axis_name="subcore")

    @pl.kernel(
        out_type=jax.ShapeDtypeStruct((n, HIDDEN), table.dtype),
        mesh=mesh,
    )
    def gather_kernel(table_hbm, i_hbm, o_hbm):
        def body(i_vmem, o_vmem):
            pltpu.sync_copy(table_hbm.at[i_vmem.at[0]], o_vmem)

        pltpu.emit_pipeline(
            body,
            grid=(n // GATHER_WINDOW,),
            in_specs=[pl.BlockSpec((1, GATHER_WINDOW), index_map=lambda i: (0, i))],
            out_specs=[pl.BlockSpec((GATHER_WINDOW, HIDDEN), index_map=lambda i: (i, 0))],
            core_axis_name=("core", "subcore"),
            dimension_semantics=(pltpu.PARALLEL,),
        )(i_hbm, o_hbm)

    return gather_kernel(table, ids2)


def _mm_body(x_ref, w_ref, b_ref, o_ref):
    o_ref[...] = (
        jnp.dot(
            x_ref[...].astype(jnp.bfloat16),
            w_ref[...],
            preferred_element_type=jnp.float32,
        )
        + b_ref[...]
    )


def _mm_body_acc(x_ref, w_ref, b_ref, _acc_ref, o_ref):
    _mm_body(x_ref, w_ref, b_ref, o_ref)


def _tc_project_chunk(x, W, b2, n_total, block_offset, out_buf):
    n_c = x.shape[0]
    grid = (n_c // MM_BLK,)
    x_spec = pl.BlockSpec((MM_BLK, HIDDEN), lambda i: (i, 0))
    w_spec = pl.BlockSpec((HIDDEN, EMBED), lambda i: (0, 0))
    b_spec = pl.BlockSpec((1, EMBED), lambda i: (0, 0))
    out_spec = pl.BlockSpec(
        (MM_BLK, EMBED), lambda i, _o=block_offset: (i + _o, 0)
    )
    out_shape = jax.ShapeDtypeStruct((n_total, EMBED), jnp.float32)
    if out_buf is None:
        return pl.pallas_call(
            _mm_body,
            grid=grid,
            in_specs=[x_spec, w_spec, b_spec],
            out_specs=out_spec,
            out_shape=out_shape,
        )(x, W, b2)
    acc_spec = pl.BlockSpec(memory_space=pl.ANY)
    return pl.pallas_call(
        _mm_body_acc,
        grid=grid,
        in_specs=[x_spec, w_spec, b_spec, acc_spec],
        out_specs=out_spec,
        out_shape=out_shape,
        input_output_aliases={3: 0},
    )(x, W, b2, out_buf)


def kernel(input_ids, table, W, b):
    B, L = input_ids.shape
    n = B * L
    n_c = n // CHUNKS
    ids_flat = input_ids.reshape(n)
    b2 = b.reshape(1, EMBED)
    W_bf = W.astype(jnp.bfloat16)
    xs = [
        _sc_gather(table, jax.lax.dynamic_slice_in_dim(ids_flat, i * n_c, n_c))
        for i in range(CHUNKS)
    ]
    out = None
    for i in range(CHUNKS):
        out = _tc_project_chunk(xs[i], W_bf, b2, n, i * (n_c // MM_BLK), out)
    return out.reshape(B, L, EMBED)


# speedup vs baseline: 1.2273x; 1.2273x over previous
import jax
import jax.numpy as jnp
import numpy as np
from jax.experimental import pallas as pl
from jax.experimental.pallas import tpu as pltpu
from jax.experimental.pallas import tpu_sc as plsc
from jax.sharding import Mesh, PartitionSpec as P

try:
    from jax import shard_map as _shard_map
except ImportError:
    from jax.experimental.shard_map import shard_map as _shard_map

HIDDEN = 128
EMBED = 1024
GATHER_WINDOW = 128
MM_BLK = 1600
CHUNKS = 4


def _sc_gather(table, ids_flat):
    n = ids_flat.shape[0]
    ids2 = ids_flat.reshape(1, n)
    mesh = plsc.VectorSubcoreMesh(core_axis_name="core", subcore_axis_name="subcore")

    @pl.kernel(
        out_type=jax.ShapeDtypeStruct((n, HIDDEN), table.dtype),
        mesh=mesh,
    )
    def gather_kernel(table_hbm, i_hbm, o_hbm):
        def body(i_vmem, o_vmem):
            pltpu.sync_copy(table_hbm.at[i_vmem.at[0]], o_vmem)

        pltpu.emit_pipeline(
            body,
            grid=(n // GATHER_WINDOW,),
            in_specs=[pl.BlockSpec((1, GATHER_WINDOW), index_map=lambda i: (0, i))],
            out_specs=[pl.BlockSpec((GATHER_WINDOW, HIDDEN), index_map=lambda i: (i, 0))],
            core_axis_name=("core", "subcore"),
            dimension_semantics=(pltpu.PARALLEL,),
        )(i_hbm, o_hbm)

    return gather_kernel(table, ids2)


def _mm_body(x_ref, w_ref, b_ref, o_ref):
    o_ref[...] = (
        jnp.dot(
            x_ref[...].astype(jnp.bfloat16),
            w_ref[...],
            preferred_element_type=jnp.float32,
        )
        + b_ref[...]
    )


def _mm_body_acc(x_ref, w_ref, b_ref, _acc_ref, o_ref):
    _mm_body(x_ref, w_ref, b_ref, o_ref)


def _tc_project_chunk(x, W, b2, n_total, block_offset, out_buf):
    n_c = x.shape[0]
    grid = (n_c // MM_BLK,)
    x_spec = pl.BlockSpec((MM_BLK, HIDDEN), lambda i: (i, 0))
    w_spec = pl.BlockSpec((HIDDEN, EMBED), lambda i: (0, 0))
    b_spec = pl.BlockSpec((1, EMBED), lambda i: (0, 0))
    out_spec = pl.BlockSpec(
        (MM_BLK, EMBED), lambda i, _o=block_offset: (i + _o, 0)
    )
    out_shape = jax.ShapeDtypeStruct((n_total, EMBED), jnp.float32)
    if out_buf is None:
        return pl.pallas_call(
            _mm_body,
            grid=grid,
            in_specs=[x_spec, w_spec, b_spec],
            out_specs=out_spec,
            out_shape=out_shape,
        )(x, W, b2)
    acc_spec = pl.BlockSpec(memory_space=pl.ANY)
    return pl.pallas_call(
        _mm_body_acc,
        grid=grid,
        in_specs=[x_spec, w_spec, b_spec, acc_spec],
        out_specs=out_spec,
        out_shape=out_shape,
        input_output_aliases={3: 0},
    )(x, W, b2, out_buf)


def _device_pipeline(input_ids, table, W, b):
    B, L = input_ids.shape
    n = B * L
    n_c = n // CHUNKS
    ids_flat = input_ids.reshape(n)
    b2 = b.reshape(1, EMBED)
    W_bf = W.astype(jnp.bfloat16)
    xs = [
        _sc_gather(table, jax.lax.dynamic_slice_in_dim(ids_flat, i * n_c, n_c))
        for i in range(CHUNKS)
    ]
    out = None
    for i in range(CHUNKS):
        out = _tc_project_chunk(xs[i], W_bf, b2, n, i * (n_c // MM_BLK), out)
    return out.reshape(B, L, EMBED)


def kernel(input_ids, table, W, b):
    devs = jax.devices()[:2]
    mesh = Mesh(np.array(devs), ("d",))
    f = _shard_map(
        _device_pipeline,
        mesh=mesh,
        in_specs=(P("d"), P(), P(), P()),
        out_specs=P("d"),
        check_vma=False,
    )
    return f(input_ids, table, W, b)
